# Initial kernel scaffold; baseline (speedup 1.0000x reference)
import jax
import jax.numpy as jnp
from jax.experimental import pallas as pl


def kernel(input_ids, core1_q, core1_scale, core1_zp, core2_q, core2_scale, core2_zp, phase_shift):
    raise NotImplementedError("write your pallas kernel here")


# speedup vs baseline: 1.3003x; 1.3003x over previous
import functools

import jax
import jax.numpy as jnp
from jax import lax
from jax.experimental import pallas as pl
from jax.experimental.pallas import tpu as pltpu
from jax.experimental.pallas import tpu_sc as plsc

_V1 = 1000
_V2 = 1000
_RANK = 64
_D1 = 12
_D2 = 11
_DM = 128
_WORDS = (_D1 * _RANK) // 4
_CHUNK = 128


def _sc_gather_fn(ntok, tok_per_w, nc):
  mesh = plsc.VectorSubcoreMesh(core_axis_name="c", subcore_axis_name="s")

  nchunks = tok_per_w // _CHUNK

  @functools.partial(
      pl.kernel,
      mesh=mesh,
      compiler_params=pltpu.CompilerParams(use_tc_tiling_on_sc=False),
      out_type=(
          jax.ShapeDtypeStruct((ntok, _WORDS), jnp.int32),
          jax.ShapeDtypeStruct((ntok, _WORDS), jnp.int32),
      ),
      scratch_types=[
          pltpu.VMEM((tok_per_w,), jnp.int32),
          pltpu.VMEM((tok_per_w,), jnp.int32),
          pltpu.VMEM((tok_per_w,), jnp.int32),
          pltpu.VMEM((2, _CHUNK, _WORDS), jnp.int32),
          pltpu.VMEM((2, _CHUNK, _WORDS), jnp.int32),
          pltpu.SemaphoreType.DMA,
          pltpu.SemaphoreType.DMA,
          pltpu.SemaphoreType.DMA,
          pltpu.SemaphoreType.DMA,
      ],
  )
  def sc_gather(ids_hbm, t1_hbm, t2_hbm, g1_hbm, g2_hbm,
                ids_v, idx1_v, idx2_v, rows1_v, rows2_v,
                gsem_a, gsem_b, ssem_a, ssem_b):
    wid = lax.axis_index("s") * nc + lax.axis_index("c")
    wbase = wid * tok_per_w
    pltpu.sync_copy(ids_hbm.at[pl.ds(wbase, tok_per_w)], ids_v)
    vmod = jnp.full((16,), _V2, jnp.int32)
    vone = jnp.full((16,), 1, jnp.int32)
    vrcp = jnp.full((16,), 1.0 / _V2, jnp.float32)
    for i in range(tok_per_w // 16):
      sl = pl.ds(i * 16, 16)
      v = ids_v[sl]
      q = (v.astype(jnp.float32) * vrcp).astype(jnp.int32)
      q = jnp.where(q * vmod > v, q - vone, q)
      q = jnp.where(q * vmod + vmod <= v, q + vone, q)
      idx1_v[sl] = q
      idx2_v[sl] = v - q * vmod
    gsems = (gsem_a, gsem_b)
    ssems = (ssem_a, ssem_b)

    def fire_gather(c):
      buf = c % 2
      isl = pl.ds(c * _CHUNK, _CHUNK)
      cp1 = pltpu.async_copy(t1_hbm.at[idx1_v.at[isl]], rows1_v.at[buf],
                             gsems[buf])
      cp2 = pltpu.async_copy(t2_hbm.at[idx2_v.at[isl]], rows2_v.at[buf],
                             gsems[buf])
      return cp1, cp2

    gathers = [None] * nchunks
    scatters = [None] * nchunks
    gathers[0] = fire_gather(0)
    for c in range(nchunks):
      buf = c % 2
      g1c, g2c = gathers[c]
      g1c.wait()
      g2c.wait()
      if c + 1 < nchunks:
        if c - 1 >= 0:
          s1p, s2p = scatters[c - 1]
          s1p.wait()
          s2p.wait()
        gathers[c + 1] = fire_gather(c + 1)
      osl = pl.ds(wbase + c * _CHUNK, _CHUNK)
      scatters[c] = (
          pltpu.async_copy(rows1_v.at[buf], g1_hbm.at[osl], ssems[buf]),
          pltpu.async_copy(rows2_v.at[buf], g2_hbm.at[osl], ssems[buf]),
      )
    for c in (nchunks - 2, nchunks - 1):
      if c >= 0:
        s1p, s2p = scatters[c]
        s1p.wait()
        s2p.wait()

  return sc_gather


def _tc_body(tb, g1_ref, g2_ref, ph_ref, par_ref, o_ref):
  s = par_ref[0]
  zp1 = par_ref[1]
  zp2 = par_ref[2]
  mult = s * jnp.cos(ph_ref[...])
  a_t = g1_ref[...].astype(jnp.float32)
  b_t = g2_ref[...].astype(jnp.float32)
  a4 = a_t.T.reshape(_D1, 8, 8, tb)
  b4 = b_t.T.reshape(_D1, 8, 8, tb)
  m4 = mult.reshape(8, 8, 1)
  bs = [b4[d2] - zp2 for d2 in range(_D2)]
  rows = []
  for d1 in range(_D1):
    a = (a4[d1] - zp1) * m4
    for d2 in range(_D2):
      f = d1 * _D2 + d2
      if f >= _DM:
        break
      b = bs[d2]
      ps = [a[k] * b[k] for k in range(8)]
      while len(ps) > 1:
        ps = [ps[i] + ps[i + 1] for i in range(0, len(ps), 2)]
      rows.append(jnp.sum(ps[0], axis=0, keepdims=True))
  out_f = jnp.concatenate(rows, axis=0)
  o_ref[...] = out_f.T


def kernel(input_ids, core1_q, core1_scale, core1_zp, core2_q, core2_scale,
           core2_zp, phase_shift):
  b, l = input_ids.shape
  ntok = b * l
  ids = input_ids.reshape(ntok).astype(jnp.int32)

  t1 = core1_q.reshape(_V1, _RANK, _D1).transpose(0, 2, 1)
  t2 = core2_q.reshape(_V2, _RANK, _D2).transpose(0, 2, 1)
  t2 = jnp.pad(t2, ((0, 0), (0, _D1 - _D2), (0, 0)))
  t1_w = lax.bitcast_convert_type(
      t1.reshape(_V1, _WORDS, 4), jnp.int32)
  t2_w = lax.bitcast_convert_type(
      t2.reshape(_V2, _WORDS, 4), jnp.int32)

  info = plsc.get_sparse_core_info()
  nw = info.num_cores * info.num_subcores
  tok_per_w = ntok // nw

  g1_w, g2_w = _sc_gather_fn(ntok, tok_per_w, info.num_cores)(ids, t1_w, t2_w)

  g1 = lax.bitcast_convert_type(g1_w, jnp.int8).reshape(ntok, 4 * _WORDS)
  g2 = lax.bitcast_convert_type(g2_w, jnp.int8).reshape(ntok, 4 * _WORDS)

  params = jnp.stack([
      (core1_scale * core2_scale).astype(jnp.float32),
      core1_zp.astype(jnp.float32),
      core2_zp.astype(jnp.float32),
      jnp.float32(0.0),
  ])
  ph = phase_shift.reshape(_RANK, 1).astype(jnp.float32)

  tb = 1024
  nblk = ntok // tb
  out = pl.pallas_call(
      functools.partial(_tc_body, tb),
      grid=(nblk,),
      in_specs=[
          pl.BlockSpec((tb, 4 * _WORDS), lambda i: (i, 0)),
          pl.BlockSpec((tb, 4 * _WORDS), lambda i: (i, 0)),
          pl.BlockSpec((_RANK, 1), lambda i: (0, 0)),
          pl.BlockSpec(memory_space=pltpu.SMEM),
      ],
      out_specs=pl.BlockSpec((tb, _DM), lambda i: (i, 0)),
      out_shape=jax.ShapeDtypeStruct((ntok, _DM), jnp.float32),
  )(g1, g2, ph, params)

  return out.reshape(b, l, _DM)


# speedup vs baseline: 3.5619x; 2.7394x over previous
import functools

import jax
import jax.numpy as jnp
from jax import lax
from jax.experimental import pallas as pl
from jax.experimental.pallas import tpu as pltpu
from jax.experimental.pallas import tpu_sc as plsc

_V1 = 1000
_V2 = 1000
_RANK = 64
_D1 = 12
_D2 = 11
_DM = 128
_WORDS = (_D1 * _RANK) // 4
_CHUNK = 128


def _sc_gather_fn(ntok, tok_per_w, nc):
  mesh = plsc.VectorSubcoreMesh(core_axis_name="c", subcore_axis_name="s")

  nchunks = tok_per_w // _CHUNK

  @functools.partial(
      pl.kernel,
      mesh=mesh,
      compiler_params=pltpu.CompilerParams(use_tc_tiling_on_sc=False),
      out_type=(
          jax.ShapeDtypeStruct((ntok, _WORDS), jnp.int32),
          jax.ShapeDtypeStruct((ntok, _WORDS), jnp.int32),
      ),
      scratch_types=[
          pltpu.VMEM((tok_per_w,), jnp.int32),
          pltpu.VMEM((tok_per_w,), jnp.int32),
          pltpu.VMEM((tok_per_w,), jnp.int32),
          pltpu.VMEM((2, _CHUNK, _WORDS), jnp.int32),
          pltpu.VMEM((2, _CHUNK, _WORDS), jnp.int32),
          pltpu.SemaphoreType.DMA,
          pltpu.SemaphoreType.DMA,
          pltpu.SemaphoreType.DMA,
          pltpu.SemaphoreType.DMA,
      ],
  )
  def sc_gather(ids_hbm, t1_hbm, t2_hbm, g1_hbm, g2_hbm,
                ids_v, idx1_v, idx2_v, rows1_v, rows2_v,
                gsem_a, gsem_b, ssem_a, ssem_b):
    wid = lax.axis_index("s") * nc + lax.axis_index("c")
    wbase = wid * tok_per_w
    pltpu.sync_copy(ids_hbm.at[pl.ds(wbase, tok_per_w)], ids_v)
    vmod = jnp.full((16,), _V2, jnp.int32)
    vone = jnp.full((16,), 1, jnp.int32)
    vrcp = jnp.full((16,), 1.0 / _V2, jnp.float32)
    for i in range(tok_per_w // 16):
      sl = pl.ds(i * 16, 16)
      v = ids_v[sl]
      q = (v.astype(jnp.float32) * vrcp).astype(jnp.int32)
      q = jnp.where(q * vmod > v, q - vone, q)
      q = jnp.where(q * vmod + vmod <= v, q + vone, q)
      idx1_v[sl] = q
      idx2_v[sl] = v - q * vmod
    gsems = (gsem_a, gsem_b)
    ssems = (ssem_a, ssem_b)

    def fire_gather(c):
      buf = c % 2
      isl = pl.ds(c * _CHUNK, _CHUNK)
      cp1 = pltpu.async_copy(t1_hbm.at[idx1_v.at[isl]], rows1_v.at[buf],
                             gsems[buf])
      cp2 = pltpu.async_copy(t2_hbm.at[idx2_v.at[isl]], rows2_v.at[buf],
                             gsems[buf])
      return cp1, cp2

    gathers = [None] * nchunks
    scatters = [None] * nchunks
    gathers[0] = fire_gather(0)
    for c in range(nchunks):
      buf = c % 2
      g1c, g2c = gathers[c]
      g1c.wait()
      g2c.wait()
      if c + 1 < nchunks:
        if c - 1 >= 0:
          s1p, s2p = scatters[c - 1]
          s1p.wait()
          s2p.wait()
        gathers[c + 1] = fire_gather(c + 1)
      osl = pl.ds(wbase + c * _CHUNK, _CHUNK)
      scatters[c] = (
          pltpu.async_copy(rows1_v.at[buf], g1_hbm.at[osl], ssems[buf]),
          pltpu.async_copy(rows2_v.at[buf], g2_hbm.at[osl], ssems[buf]),
      )
    for c in (nchunks - 2, nchunks - 1):
      if c >= 0:
        s1p, s2p = scatters[c]
        s1p.wait()
        s2p.wait()

  return sc_gather


def _tc_body(tb, g1_ref, g2_ref, ph_ref, par_ref, o_ref):
  s = par_ref[0]
  zp1 = par_ref[1]
  zp2 = par_ref[2]
  mult = (s * jnp.cos(ph_ref[...])).reshape(4, 16, 1)
  w1 = g1_ref[...].T
  w2 = g2_ref[...].T
  a_pl = []
  b_pl = []
  for b in range(4):
    p1 = jnp.right_shift(jnp.left_shift(w1, 24 - 8 * b), 24)
    p13 = p1.astype(jnp.float32).reshape(_D1, 16, tb)
    a_pl.append([(p13[d] - zp1) * mult[b] for d in range(_D1)])
    p2 = jnp.right_shift(jnp.left_shift(w2, 24 - 8 * b), 24)
    p23 = p2.astype(jnp.float32).reshape(_D1, 16, tb)
    b_pl.append([p23[d] - zp2 for d in range(_D2)])
  rows = []
  for d1 in range(_D1):
    for d2 in range(_D2):
      f = d1 * _D2 + d2
      if f >= _DM:
        break
      ps = [a_pl[b][d1] * b_pl[b][d2] for b in range(4)]
      acc = (ps[0] + ps[1]) + (ps[2] + ps[3])
      rows.append(jnp.sum(acc, axis=0, keepdims=True))
  out_f = jnp.concatenate(rows, axis=0)
  o_ref[...] = out_f.T


def kernel(input_ids, core1_q, core1_scale, core1_zp, core2_q, core2_scale,
           core2_zp, phase_shift):
  b, l = input_ids.shape
  ntok = b * l
  ids = input_ids.reshape(ntok).astype(jnp.int32)

  t1 = core1_q.reshape(_V1, _RANK, _D1).transpose(0, 2, 1)
  t2 = core2_q.reshape(_V2, _RANK, _D2).transpose(0, 2, 1)
  t2 = jnp.pad(t2, ((0, 0), (0, _D1 - _D2), (0, 0)))
  t1_w = lax.bitcast_convert_type(
      t1.reshape(_V1, _WORDS, 4), jnp.int32)
  t2_w = lax.bitcast_convert_type(
      t2.reshape(_V2, _WORDS, 4), jnp.int32)

  info = plsc.get_sparse_core_info()
  nw = info.num_cores * info.num_subcores
  tok_per_w = ntok // nw

  g1_w, g2_w = _sc_gather_fn(ntok, tok_per_w, info.num_cores)(ids, t1_w, t2_w)

  params = jnp.stack([
      (core1_scale * core2_scale).astype(jnp.float32),
      core1_zp.astype(jnp.float32),
      core2_zp.astype(jnp.float32),
      jnp.float32(0.0),
  ])
  ph = phase_shift.reshape(16, 4).T.astype(jnp.float32)

  tb = 1024
  nblk = ntok // tb
  out = pl.pallas_call(
      functools.partial(_tc_body, tb),
      grid=(nblk,),
      in_specs=[
          pl.BlockSpec((tb, _WORDS), lambda i: (i, 0)),
          pl.BlockSpec((tb, _WORDS), lambda i: (i, 0)),
          pl.BlockSpec((4, 16), lambda i: (0, 0)),
          pl.BlockSpec(memory_space=pltpu.SMEM),
      ],
      out_specs=pl.BlockSpec((tb, _DM), lambda i: (i, 0)),
      out_shape=jax.ShapeDtypeStruct((ntok, _DM), jnp.float32),
  )(g1_w, g2_w, ph, params)

  return out.reshape(b, l, _DM)
